# Initial kernel scaffold; baseline (speedup 1.0000x reference)
import jax
import jax.numpy as jnp
from jax.experimental import pallas as pl


def kernel(logits, ranks):
    raise NotImplementedError("write your pallas kernel here")


# speedup vs baseline: 12.8149x; 12.8149x over previous
import jax
import jax.numpy as jnp
from jax.experimental import pallas as pl

_N = 100000
_MAX_RANK = 20
_ROWS = 784
_COLS = 128
_PAD = _ROWS * _COLS
_KEY_MUL = 1 << 17
_NEG = -(1 << 30)


def _loss_kernel(logits_ref, ranks_ref, out_ref):
    logits = logits_ref[...]
    ranks = ranks_ref[...]
    row = jax.lax.broadcasted_iota(jnp.int32, (_ROWS, _COLS), 0)
    col = jax.lax.broadcasted_iota(jnp.int32, (_ROWS, _COLS), 1)
    idx = row * _COLS + col
    key = ranks * _KEY_MUL + idx
    valid = ranks >= 0

    n_labels = jnp.max(ranks)

    sel_keys = []
    sel_logits = []
    cur = key
    for _ in range(_MAX_RANK):
        mk = jnp.max(cur)
        hit = cur == mk
        lg = jnp.max(jnp.where(hit, logits, -jnp.inf))
        sel_keys.append(mk)
        sel_logits.append(lg)
        cur = jnp.where(hit, _NEG, cur)

    thr = jnp.int32(0)
    low_lg = jnp.float32(0.0)
    for j in range(_MAX_RANK):
        pick = n_labels - 1 == j
        thr = jnp.where(pick, sel_keys[j], thr)
        low_lg = jnp.where(pick, sel_logits[j], low_lg)

    rem_lg = jnp.max(jnp.where(valid & (key < thr), logits, -jnp.inf))

    probs = [jax.nn.sigmoid(lg) for lg in sel_logits]
    t = jax.nn.sigmoid(low_lg) - jax.nn.sigmoid(rem_lg)
    loss1 = jnp.maximum(t, 0.0)

    correct = jnp.int32(0)
    for i in range(_MAX_RANK - 1):
        for j in range(i + 1, _MAX_RANK):
            c = (j < n_labels) & (probs[i] > probs[j])
            correct = correct + c.astype(jnp.int32)
    total = n_labels * (n_labels - 1) // 2
    loss2 = jnp.where(
        total > 0,
        1.0 - correct.astype(jnp.float32) / jnp.maximum(total, 1).astype(jnp.float32),
        jnp.float32(0.0),
    )
    out = jnp.where(n_labels != 0, loss1 + loss2, jnp.float32(0.0))
    out_ref[...] = jnp.broadcast_to(out, (1, 1))


def kernel(logits, ranks):
    logits_p = jnp.zeros((_PAD,), jnp.float32).at[:_N].set(logits)
    ranks_p = jnp.full((_PAD,), -1, jnp.int32).at[:_N].set(ranks)
    out = pl.pallas_call(
        _loss_kernel,
        out_shape=jax.ShapeDtypeStruct((1, 1), jnp.float32),
    )(logits_p.reshape(_ROWS, _COLS), ranks_p.reshape(_ROWS, _COLS))
    return out[0, 0]
